# baseline (device time: 290514 ns/iter reference)
import jax
import jax.numpy as jnp
from jax import lax
from jax.experimental import pallas as pl
from jax.experimental.pallas import tpu as pltpu

K = 32
J = K // 2


def kernel(x):
    m, n = x.shape
    assert m % K == 0
    c = m // K

    def body(x_ref, out_ref, recv_ref, xbb_ref,
             xs_sems, xr_sems, fs_sems, yr_sems,
             fv, bv, rv, ov, cfl_sems, cst_sems, cpx_sems, cpr_sems,
             cpo_sems):
        mx = lax.axis_index("x")
        my = lax.axis_index("y")
        mz = lax.axis_index("z")
        xpeer = (1 - mx, my, mz)
        ypeer = (mx, 1 - my, mz)

        barrier = pltpu.get_barrier_semaphore()
        for nbr in (xpeer, ypeer):
            pl.semaphore_signal(barrier, inc=1, device_id=nbr,
                                device_id_type=pl.DeviceIdType.MESH)
        pl.semaphore_wait(barrier, 2)

        def chunk(ref, t):
            return ref.at[pl.ds(t * c, c), :]

        def direct_id(j):
            return 2 * j + my

        def fwd_id(j):
            return 2 * j + (1 - my)

        xsends = []
        stores = [None, None]
        for j in range(J + 1):
            s = j % 2
            if j < J:
                t = direct_id(j)
                fl = pltpu.make_async_copy(chunk(x_ref, t), fv.at[s],
                                           cfl_sems.at[s])
                fl.start()
            if j > 0:
                stores[1 - s].wait()
                tp = direct_id(j - 1)
                rdma = pltpu.make_async_remote_copy(
                    src_ref=chunk(xbb_ref, tp),
                    dst_ref=chunk(recv_ref, tp),
                    send_sem=xs_sems.at[j - 1],
                    recv_sem=xr_sems.at[j - 1],
                    device_id=xpeer,
                    device_id_type=pl.DeviceIdType.MESH,
                )
                rdma.start()
                xsends.append(rdma)
            if j < J:
                fl.wait()
                bv[s] = fv[s].astype(jnp.bfloat16)
                st = pltpu.make_async_copy(bv.at[s], chunk(xbb_ref, t),
                                           cst_sems.at[s])
                st.start()
                stores[s] = st

        yrecvs = []
        for j in range(J):
            t = fwd_id(j)
            yrecvs.append(pltpu.make_async_remote_copy(
                src_ref=chunk(recv_ref, t),
                dst_ref=chunk(recv_ref, t),
                send_sem=fs_sems.at[j],
                recv_sem=yr_sems.at[j],
                device_id=ypeer,
                device_id_type=pl.DeviceIdType.MESH,
            ))

        items = []
        for j in range(J):
            items.append(("d", j))
            items.append(("f", j))

        fwds = []
        pending = None
        store_chunk = [None, None]
        for idx, (kind, j) in enumerate(items):
            s = idx % 2
            if kind == "d":
                xsends[j].wait_recv()
                t = direct_id(j)
                fwd = pltpu.make_async_remote_copy(
                    src_ref=chunk(recv_ref, t),
                    dst_ref=chunk(recv_ref, t),
                    send_sem=fs_sems.at[j],
                    recv_sem=yr_sems.at[j],
                    device_id=ypeer,
                    device_id_type=pl.DeviceIdType.MESH,
                )
                fwd.start()
                fwds.append(fwd)
            else:
                yrecvs[j].wait_recv()
                t = fwd_id(j)
            cx = pltpu.make_async_copy(chunk(x_ref, t), fv.at[s],
                                       cpx_sems.at[s])
            cr = pltpu.make_async_copy(chunk(recv_ref, t), rv.at[s],
                                       cpr_sems.at[s])
            cx.start()
            cr.start()
            if pending is not None:
                ps, pt = pending
                pltpu.make_async_copy(chunk(x_ref, pt), fv.at[ps],
                                      cpx_sems.at[ps]).wait()
                pltpu.make_async_copy(chunk(recv_ref, pt), rv.at[ps],
                                      cpr_sems.at[ps]).wait()
                if store_chunk[ps] is not None:
                    pltpu.make_async_copy(
                        ov.at[ps], chunk(out_ref, store_chunk[ps]),
                        cpo_sems.at[ps]).wait()
                ov[ps] = fv[ps].astype(jnp.bfloat16) + rv[ps]
                co = pltpu.make_async_copy(ov.at[ps], chunk(out_ref, pt),
                                           cpo_sems.at[ps])
                co.start()
                store_chunk[ps] = pt
            pending = (s, t)
        ps, pt = pending
        pltpu.make_async_copy(chunk(x_ref, pt), fv.at[ps],
                              cpx_sems.at[ps]).wait()
        pltpu.make_async_copy(chunk(recv_ref, pt), rv.at[ps],
                              cpr_sems.at[ps]).wait()
        if store_chunk[ps] is not None:
            pltpu.make_async_copy(ov.at[ps], chunk(out_ref, store_chunk[ps]),
                                  cpo_sems.at[ps]).wait()
        ov[ps] = fv[ps].astype(jnp.bfloat16) + rv[ps]
        co = pltpu.make_async_copy(ov.at[ps], chunk(out_ref, pt),
                                   cpo_sems.at[ps])
        co.start()
        co.wait()
        if store_chunk[1 - ps] is not None:
            pltpu.make_async_copy(ov.at[1 - ps],
                                  chunk(out_ref, store_chunk[1 - ps]),
                                  cpo_sems.at[1 - ps]).wait()

        for j in range(J):
            xsends[j].wait_send()
            fwds[j].wait_send()

    out, _recv, _xbb = pl.pallas_call(
        body,
        out_shape=(
            jax.ShapeDtypeStruct((m, n), jnp.bfloat16),
            jax.ShapeDtypeStruct((m, n), jnp.bfloat16),
            jax.ShapeDtypeStruct((m, n), jnp.bfloat16),
        ),
        in_specs=[pl.BlockSpec(memory_space=pl.ANY)],
        out_specs=(
            pl.BlockSpec(memory_space=pl.ANY),
            pl.BlockSpec(memory_space=pl.ANY),
            pl.BlockSpec(memory_space=pl.ANY),
        ),
        scratch_shapes=[
            pltpu.SemaphoreType.DMA((J,)),
            pltpu.SemaphoreType.DMA((J,)),
            pltpu.SemaphoreType.DMA((J,)),
            pltpu.SemaphoreType.DMA((J,)),
            pltpu.MemorySpace.VMEM((2, c, n), jnp.float32),
            pltpu.MemorySpace.VMEM((2, c, n), jnp.bfloat16),
            pltpu.MemorySpace.VMEM((2, c, n), jnp.bfloat16),
            pltpu.MemorySpace.VMEM((2, c, n), jnp.bfloat16),
            pltpu.SemaphoreType.DMA((2,)),
            pltpu.SemaphoreType.DMA((2,)),
            pltpu.SemaphoreType.DMA((2,)),
            pltpu.SemaphoreType.DMA((2,)),
            pltpu.SemaphoreType.DMA((2,)),
        ],
        compiler_params=pltpu.CompilerParams(collective_id=0),
    )(x)
    return out
